# baseline (speedup 1.0000x reference)
import functools

import jax
import jax.numpy as jnp
import numpy as np
from jax.experimental import pallas as pl
from jax.experimental.pallas import tpu as pltpu

N = 16
SCALE = 3.0
LP = 10
LD = 4
E = N * N * N
P = 32


def _enc_consts(ncols, L):
    k = np.arange(ncols)
    g = (k - 3) // 6
    c = (k - 3) % 6
    valid = (k >= 3) & (k < 3 + 6 * L)
    scale = np.where(valid, 2.0 ** np.maximum(g, 0), 0.0)
    mid = (k < 3).astype(np.float32)
    msin = (valid & (c < 3)).astype(np.float32)
    mcos = (valid & (c >= 3)).astype(np.float32)
    sel = np.zeros((3, ncols), np.float32)
    sel[k % 3, k] = 1.0
    return (sel, scale.astype(np.float32), mid, msin.astype(np.float32),
            mcos.astype(np.float32))


_SX, _SCX, _MIDX, _MSINX, _MCOSX = _enc_consts(64, LP)
_SD, _SCD, _MIDD, _MSIND, _MCOSD = _enc_consts(32, LD)
_SELC = np.array([1, 1, 1, 0, 0, 0, 0, 0], np.float32)
_E3 = np.array([0, 0, 0, 1, 0, 0, 0, 0], np.float32)


def _pack_consts():
    ct = np.zeros((16, 64), np.float32)
    ct[0:3, :] = _SX
    ct[3, :] = _SCX
    ct[4, :] = _MIDX
    ct[5, :] = _MSINX
    ct[6, :] = _MCOSX
    ct[8:11, 0:32] = _SD
    ct[11, 0:32] = _SCD
    ct[12, 0:32] = _MIDD
    ct[13, 0:32] = _MSIND
    ct[14, 0:32] = _MCOSD
    ct[15, 0:8] = _SELC
    ct[15, 8:16] = _E3
    return ct


_CT = _pack_consts()


_TWO_PI = float(2.0 * np.pi)


def _rr(a):
    y = a * (1.0 / _TWO_PI)
    return (y - jnp.round(y)) * _TWO_PI


def _mlp_body(te_ref, tc_ref, ct_ref, xd_ref, w1_ref, w2_ref, w3_ref, w4_ref,
              w5_ref, b1_ref, b2_ref, b3_ref, b4_ref, b5_ref, out_ref):
    t = pl.program_id(0)

    @pl.when(tc_ref[t] > 0)
    def _():
        ct = ct_ref[...]
        xd = xd_ref[0]
        xb = xd[:, 0:3]
        db = xd[:, 3:6]

        X = jnp.dot(xb, ct[0:3, :], preferred_element_type=jnp.float32, precision=jax.lax.Precision.HIGHEST)
        A = _rr(X * ct[3:4, :])
        embx = ct[4:5, :] * X + ct[5:6, :] * jnp.sin(A) + ct[6:7, :] * jnp.cos(A)

        D = jnp.dot(db, ct[8:11, 0:32], preferred_element_type=jnp.float32, precision=jax.lax.Precision.HIGHEST)
        Ad = _rr(D * ct[11:12, 0:32])
        embd = (ct[12:13, 0:32] * D + ct[13:14, 0:32] * jnp.sin(Ad)
                + ct[14:15, 0:32] * jnp.cos(Ad))

        h1 = jax.nn.relu(
            jnp.dot(embx, w1_ref[0], preferred_element_type=jnp.float32, precision=jax.lax.Precision.HIGHEST)
            + b1_ref[0])
        h2 = jax.nn.relu(
            jnp.dot(h1, w2_ref[0], preferred_element_type=jnp.float32, precision=jax.lax.Precision.HIGHEST)
            + b2_ref[0])
        density = h2[:, 32:33]
        h3 = (jnp.dot(h2[:, 0:32], w3_ref[0], preferred_element_type=jnp.float32, precision=jax.lax.Precision.HIGHEST)
              + b3_ref[0])
        h4 = jax.nn.relu(
            jnp.dot(h3, w4_ref[0, 0:32, :], preferred_element_type=jnp.float32, precision=jax.lax.Precision.HIGHEST)
            + jnp.dot(embd[:, 0:27], w4_ref[0, 32:59, :],
                      preferred_element_type=jnp.float32, precision=jax.lax.Precision.HIGHEST)
            + b4_ref[0])
        c8 = jax.nn.sigmoid(
            jnp.dot(h4, w5_ref[0], preferred_element_type=jnp.float32, precision=jax.lax.Precision.HIGHEST)
            + b5_ref[0])
        out_ref[0] = c8 * ct[15:16, 0:8] + density * ct[15:16, 8:16]


@jax.jit
def kernel(x, d, layer1_w, layer1_b, layer2_w, layer2_b, layer3_w, layer3_b,
           layer4_w, layer4_b, layer5_w, layer5_b):
    B = x.shape[0]
    T_MAX = E + B // P

    mask = ((jnp.abs(x[:, 0]) < SCALE / 2) & (jnp.abs(x[:, 1]) < SCALE / 2)
            & (jnp.abs(x[:, 2]) < SCALE / 2))
    idx = jnp.clip((x / (SCALE / N) + N / 2).astype(jnp.int32), 0, N - 1)
    e = (idx[:, 0] * N + idx[:, 1]) * N + idx[:, 2]
    key = jnp.where(mask, e, E)
    order = jnp.argsort(key)
    skey = key[order]
    counts = jnp.bincount(key, length=E + 1)[:E]
    off = jnp.concatenate([jnp.zeros((1,), counts.dtype), jnp.cumsum(counts)])[:E]
    ntiles = (counts + P - 1) // P
    tbase = jnp.concatenate([jnp.zeros((1,), ntiles.dtype), jnp.cumsum(ntiles)])
    t_real = tbase[E]
    tbase = tbase[:E]

    i = jnp.arange(B)
    valid = skey < E
    r = i - off[jnp.clip(skey, 0, E - 1)]
    t_of = tbase[jnp.clip(skey, 0, E - 1)] + r // P
    starts = valid & (r % P == 0)
    scat = jnp.where(starts, t_of, T_MAX)
    tile_expert = jnp.zeros((T_MAX,), jnp.int32).at[scat].set(
        skey.astype(jnp.int32), mode='drop')
    tile_count = jnp.zeros((T_MAX,), jnp.int32).at[scat].set(
        jnp.minimum(counts[jnp.clip(skey, 0, E - 1)] - r, P).astype(jnp.int32),
        mode='drop')
    tile_start = jnp.zeros((T_MAX,), jnp.int32).at[scat].set(
        i.astype(jnp.int32), mode='drop')
    last_e = jnp.maximum(jnp.max(jnp.where(valid, skey, -1)), 0).astype(jnp.int32)
    in_real = jnp.arange(T_MAX) < t_real
    tile_expert = jnp.where(in_real, tile_expert, last_e)

    slot = jnp.arange(T_MAX * P)
    ts = slot // P
    js = slot % P
    pos = tile_start[ts] + js
    in_rng = js < tile_count[ts]
    pidx = jnp.where(in_rng, order[jnp.clip(pos, 0, B - 1)], B)
    xd = jnp.concatenate([x, d], axis=1)
    xd = jnp.concatenate([xd, jnp.zeros((1, 6), x.dtype)], axis=0)
    xd_tiles = xd[pidx].reshape(T_MAX, P, 6)

    w1 = layer1_w.reshape(E, 63, 32)
    w1 = jnp.concatenate([w1, jnp.zeros((E, 1, 32), w1.dtype)], axis=1)
    w2 = layer2_w.reshape(E, 32, 33)
    w3 = layer3_w.reshape(E, 32, 32)
    w4 = layer4_w.reshape(E, 59, 32)
    w5 = jnp.pad(layer5_w.reshape(E, 32, 3), ((0, 0), (0, 0), (0, 5)))
    b1 = layer1_b.reshape(E, 1, 32)
    b2 = layer2_b.reshape(E, 1, 33)
    b3 = layer3_b.reshape(E, 1, 32)
    b4 = layer4_b.reshape(E, 1, 32)
    b5 = jnp.pad(layer5_b.reshape(E, 1, 3), ((0, 0), (0, 0), (0, 5)))

    def wmap(t, te, tc):
        return (te[t], 0, 0)

    def xmap(t, te, tc):
        return (t, 0, 0)

    grid_spec = pltpu.PrefetchScalarGridSpec(
        num_scalar_prefetch=2,
        grid=(T_MAX,),
        in_specs=[
            pl.BlockSpec((16, 64), lambda t, te, tc: (0, 0)),
            pl.BlockSpec((1, P, 6), xmap),
            pl.BlockSpec((1, 64, 32), wmap),
            pl.BlockSpec((1, 32, 33), wmap),
            pl.BlockSpec((1, 32, 32), wmap),
            pl.BlockSpec((1, 59, 32), wmap),
            pl.BlockSpec((1, 32, 8), wmap),
            pl.BlockSpec((1, 1, 32), wmap),
            pl.BlockSpec((1, 1, 33), wmap),
            pl.BlockSpec((1, 1, 32), wmap),
            pl.BlockSpec((1, 1, 32), wmap),
            pl.BlockSpec((1, 1, 8), wmap),
        ],
        out_specs=pl.BlockSpec((1, P, 8), xmap),
    )
    out_tiles = pl.pallas_call(
        _mlp_body,
        grid_spec=grid_spec,
        out_shape=jax.ShapeDtypeStruct((T_MAX, P, 8), jnp.float32),
    )(tile_expert, tile_count, jnp.asarray(_CT), xd_tiles, w1, w2, w3, w4, w5,
      b1, b2, b3, b4, b5)

    flat = out_tiles.reshape(T_MAX * P, 8)
    point_slot = jnp.zeros((B + 1,), jnp.int32).at[pidx].set(
        slot.astype(jnp.int32), mode='drop')[:B]
    res = flat[point_slot]
    color = jnp.where(mask[:, None], res[:, 0:3], 0.0)
    sigma = jnp.where(mask, res[:, 3], 0.0)
    return (color, sigma)


# speedup vs baseline: 1.5128x; 1.5128x over previous
import jax
import jax.numpy as jnp
import numpy as np
from jax.experimental import pallas as pl
from jax.experimental.pallas import tpu as pltpu

N = 16
SCALE = 3.0
LP = 10
LD = 4
E = N * N * N
P = 32
U = 4

_W1, _B1 = 0, 64
_W2, _B2 = 65, 97
_W3, _B3 = 98, 130
_W4, _B4 = 131, 190
_W5, _B5 = 191, 223
_RT = 224


def _enc_consts(ncols, L):
    k = np.arange(ncols)
    g = (k - 3) // 6
    c = (k - 3) % 6
    valid = (k >= 3) & (k < 3 + 6 * L)
    scale = np.where(valid, 2.0 ** np.maximum(g, 0), 0.0)
    mid = (k < 3).astype(np.float32)
    msin = (valid & (c < 3)).astype(np.float32)
    mcos = (valid & (c >= 3)).astype(np.float32)
    sel = np.zeros((3, ncols), np.float32)
    sel[k % 3, k] = 1.0
    return sel, scale.astype(np.float32), mid, msin.astype(np.float32), mcos.astype(np.float32)


def _pack_consts():
    sx, scx, midx, msinx, mcosx = _enc_consts(64, LP)
    sd, scd, midd, msind, mcosd = _enc_consts(32, LD)
    ct = np.zeros((16, 64), np.float32)
    ct[0:3, :] = sx
    ct[3, :] = scx
    ct[4, :] = midx
    ct[5, :] = msinx
    ct[6, :] = mcosx
    ct[8:11, 0:32] = sd
    ct[11, 0:32] = scd
    ct[12, 0:32] = midd
    ct[13, 0:32] = msind
    ct[14, 0:32] = mcosd
    ct[15, 0:8] = [1, 1, 1, 0, 0, 0, 0, 0]
    ct[15, 8:16] = [0, 0, 0, 1, 0, 0, 0, 0]
    return ct


_CT = _pack_consts()
_TWO_PI = float(2.0 * np.pi)
_HI = jax.lax.Precision.HIGHEST


def _rr(a):
    y = a * (1.0 / _TWO_PI)
    return (y - jnp.round(y)) * _TWO_PI


def _mlp_body(te_ref, ts_ref, tc_ref, ct_ref, xs_ref, *rest):
    w_refs = rest[:U]
    out_ref = rest[U]
    s = pl.program_id(0)
    ct = ct_ref[...]

    for u in range(U):
        t = s * U + u
        w = w_refs[u]

        @pl.when(tc_ref[t] > 0)
        def _(u=u, w=w, t=t):
            xd = xs_ref[pl.ds(ts_ref[t], P), :]
            xb = xd[:, 0:3]
            db = xd[:, 3:6]

            X = jnp.dot(xb, ct[0:3, :], preferred_element_type=jnp.float32,
                        precision=_HI)
            A = _rr(X * ct[3:4, :])
            embx = ct[4:5, :] * X + ct[5:6, :] * jnp.sin(A) + ct[6:7, :] * jnp.cos(A)

            D = jnp.dot(db, ct[8:11, 0:32], preferred_element_type=jnp.float32,
                        precision=_HI)
            Ad = _rr(D * ct[11:12, 0:32])
            embd = (ct[12:13, 0:32] * D + ct[13:14, 0:32] * jnp.sin(Ad)
                    + ct[14:15, 0:32] * jnp.cos(Ad))

            h1 = jax.nn.relu(
                jnp.dot(embx, w[0, _W1:_W1 + 64, 0:32],
                        preferred_element_type=jnp.float32, precision=_HI)
                + w[0, _B1:_B1 + 1, 0:32])
            h2 = jax.nn.relu(
                jnp.dot(h1, w[0, _W2:_W2 + 32, 0:33],
                        preferred_element_type=jnp.float32, precision=_HI)
                + w[0, _B2:_B2 + 1, 0:33])
            density = h2[:, 32:33]
            h3 = (jnp.dot(h2[:, 0:32], w[0, _W3:_W3 + 32, 0:32],
                          preferred_element_type=jnp.float32, precision=_HI)
                  + w[0, _B3:_B3 + 1, 0:32])
            h4 = jax.nn.relu(
                jnp.dot(h3, w[0, _W4:_W4 + 32, 0:32],
                        preferred_element_type=jnp.float32, precision=_HI)
                + jnp.dot(embd[:, 0:27], w[0, _W4 + 32:_W4 + 59, 0:32],
                          preferred_element_type=jnp.float32, precision=_HI)
                + w[0, _B4:_B4 + 1, 0:32])
            c8 = jax.nn.sigmoid(
                jnp.dot(h4, w[0, _W5:_W5 + 32, 0:8],
                        preferred_element_type=jnp.float32, precision=_HI)
                + w[0, _B5:_B5 + 1, 0:8])
            out_ref[u] = c8 * ct[15:16, 0:8] + density * ct[15:16, 8:16]


@jax.jit
def kernel(x, d, layer1_w, layer1_b, layer2_w, layer2_b, layer3_w, layer3_b,
           layer4_w, layer4_b, layer5_w, layer5_b):
    B = x.shape[0]
    T_MAX = -(-(E + B // P) // U) * U

    mask = ((jnp.abs(x[:, 0]) < SCALE / 2) & (jnp.abs(x[:, 1]) < SCALE / 2)
            & (jnp.abs(x[:, 2]) < SCALE / 2))
    idx = jnp.clip((x / (SCALE / N) + N / 2).astype(jnp.int32), 0, N - 1)
    e = (idx[:, 0] * N + idx[:, 1]) * N + idx[:, 2]
    key = jnp.where(mask, e, E)
    order = jnp.argsort(key)
    skey = key[order]
    kc = jnp.clip(key, 0, E - 1)
    counts = jnp.bincount(key, length=E + 1)[:E]
    off = jnp.concatenate([jnp.zeros((1,), counts.dtype), jnp.cumsum(counts)])[:E]
    ntiles = (counts + P - 1) // P
    tbase = jnp.concatenate([jnp.zeros((1,), ntiles.dtype), jnp.cumsum(ntiles)])
    t_real = tbase[E]
    tbase = tbase[:E]

    i = jnp.arange(B)
    skc = jnp.clip(skey, 0, E - 1)
    valid = skey < E
    r = i - off[skc]
    t_of = tbase[skc] + r // P
    starts = valid & (r % P == 0)
    scat = jnp.where(starts, t_of, T_MAX)
    tile_expert = jnp.zeros((T_MAX,), jnp.int32).at[scat].set(
        skey.astype(jnp.int32), mode='drop')
    tile_count = jnp.zeros((T_MAX,), jnp.int32).at[scat].set(
        jnp.minimum(counts[skc] - r, P).astype(jnp.int32), mode='drop')
    tile_start = jnp.zeros((T_MAX,), jnp.int32).at[scat].set(
        i.astype(jnp.int32), mode='drop')
    last_e = jnp.maximum(jnp.max(jnp.where(valid, skey, -1)), 0).astype(jnp.int32)
    in_real = jnp.arange(T_MAX) < t_real
    tile_expert = jnp.where(in_real, tile_expert, last_e)

    xd = jnp.concatenate([x, d], axis=1)[order]
    xs = jnp.concatenate([xd, jnp.zeros((P, 6), x.dtype)], axis=0)

    inv = jnp.zeros((B,), jnp.int32).at[order].set(i.astype(jnp.int32))
    rq = inv - off[kc].astype(jnp.int32)
    point_slot = ((tbase[kc].astype(jnp.int32) + rq // P) * P + rq % P)
    point_slot = jnp.where(mask, point_slot, 0)

    def pad33(a, rows):
        return jnp.pad(a.reshape(E, rows, -1),
                       ((0, 0), (0, 0), (0, 33 - a.shape[-1])))

    w_all = jnp.concatenate([
        pad33(layer1_w, 63), jnp.zeros((E, 1, 33), jnp.float32), pad33(layer1_b, 1),
        pad33(layer2_w, 32), pad33(layer2_b, 1),
        pad33(layer3_w, 32), pad33(layer3_b, 1),
        pad33(layer4_w, 59), pad33(layer4_b, 1),
        pad33(layer5_w, 32), pad33(layer5_b, 1),
    ], axis=1)

    def wmap(u):
        return lambda s, te, ts, tc: (te[s * U + u], 0, 0)

    grid_spec = pltpu.PrefetchScalarGridSpec(
        num_scalar_prefetch=3,
        grid=(T_MAX // U,),
        in_specs=[
            pl.BlockSpec((16, 64), lambda s, te, ts, tc: (0, 0)),
            pl.BlockSpec((B + P, 6), lambda s, te, ts, tc: (0, 0)),
        ] + [pl.BlockSpec((1, _RT, 33), wmap(u)) for u in range(U)],
        out_specs=pl.BlockSpec((U, P, 8), lambda s, te, ts, tc: (s, 0, 0)),
    )
    out_tiles = pl.pallas_call(
        _mlp_body,
        grid_spec=grid_spec,
        out_shape=jax.ShapeDtypeStruct((T_MAX, P, 8), jnp.float32),
    )(tile_expert, tile_start, tile_count, jnp.asarray(_CT), xs,
      *([w_all] * U))

    res = out_tiles.reshape(T_MAX * P, 8)[point_slot]
    color = jnp.where(mask[:, None], res[:, 0:3], 0.0)
    sigma = jnp.where(mask, res[:, 3], 0.0)
    return (color, sigma)


# speedup vs baseline: 1.6361x; 1.0815x over previous
import jax
import jax.numpy as jnp
import numpy as np
from jax.experimental import pallas as pl
from jax.experimental.pallas import tpu as pltpu

N = 16
SCALE = 3.0
LP = 10
LD = 4
E = N * N * N
P = 32
U = 4

_W1, _B1 = 0, 64
_W2, _B2 = 65, 97
_W3, _B3 = 98, 130
_W4, _B4 = 131, 190
_W5, _B5 = 191, 223
_RT = 224


def _enc_consts(ncols, L):
    k = np.arange(ncols)
    g = (k - 3) // 6
    c = (k - 3) % 6
    valid = (k >= 3) & (k < 3 + 6 * L)
    scale = np.where(valid, 2.0 ** np.maximum(g, 0), 0.0)
    mid = (k < 3).astype(np.float32)
    msin = (valid & (c < 3)).astype(np.float32)
    mcos = (valid & (c >= 3)).astype(np.float32)
    sel = np.zeros((3, ncols), np.float32)
    sel[k % 3, k] = 1.0
    return sel, scale.astype(np.float32), mid, msin.astype(np.float32), mcos.astype(np.float32)


def _pack_consts():
    sx, scx, midx, msinx, mcosx = _enc_consts(64, LP)
    sd, scd, midd, msind, mcosd = _enc_consts(32, LD)
    ct = np.zeros((16, 64), np.float32)
    ct[0:3, :] = sx
    ct[3, :] = scx
    ct[4, :] = midx
    ct[5, :] = msinx
    ct[6, :] = mcosx
    ct[8:11, 0:32] = sd
    ct[11, 0:32] = scd
    ct[12, 0:32] = midd
    ct[13, 0:32] = msind
    ct[14, 0:32] = mcosd
    ct[15, 0:8] = [1, 1, 1, 0, 0, 0, 0, 0]
    ct[15, 8:16] = [0, 0, 0, 1, 0, 0, 0, 0]
    return ct


_CT = _pack_consts()
_TWO_PI = float(2.0 * np.pi)
_HI = jax.lax.Precision.HIGHEST


def _rr(a):
    y = a * (1.0 / _TWO_PI)
    return (y - jnp.round(y)) * _TWO_PI


def _mlp_body(te_ref, tc_ref, ct_ref, xt_ref, *rest):
    w_refs = rest[:U]
    out_ref = rest[U]
    s = pl.program_id(0)
    ct = ct_ref[...]

    @pl.when(tc_ref[s] > 0)
    def _():
        for u in range(U):
            w = w_refs[u]
            xd = xt_ref[u]
            xb = xd[:, 0:3]
            db = xd[:, 3:6]

            X = jnp.dot(xb, ct[0:3, :], preferred_element_type=jnp.float32,
                        precision=_HI)
            A = _rr(X * ct[3:4, :])
            embx = ct[4:5, :] * X + ct[5:6, :] * jnp.sin(A) + ct[6:7, :] * jnp.cos(A)

            D = jnp.dot(db, ct[8:11, 0:32], preferred_element_type=jnp.float32,
                        precision=_HI)
            Ad = _rr(D * ct[11:12, 0:32])
            embd = (ct[12:13, 0:32] * D + ct[13:14, 0:32] * jnp.sin(Ad)
                    + ct[14:15, 0:32] * jnp.cos(Ad))

            h1 = jax.nn.relu(
                jnp.dot(embx, w[0, _W1:_W1 + 64, 0:32],
                        preferred_element_type=jnp.float32, precision=_HI)
                + w[0, _B1:_B1 + 1, 0:32])
            h2 = jax.nn.relu(
                jnp.dot(h1, w[0, _W2:_W2 + 32, 0:33],
                        preferred_element_type=jnp.float32, precision=_HI)
                + w[0, _B2:_B2 + 1, 0:33])
            density = h2[:, 32:33]
            h3 = (jnp.dot(h2[:, 0:32], w[0, _W3:_W3 + 32, 0:32],
                          preferred_element_type=jnp.float32, precision=_HI)
                  + w[0, _B3:_B3 + 1, 0:32])
            h4 = jax.nn.relu(
                jnp.dot(h3, w[0, _W4:_W4 + 32, 0:32],
                        preferred_element_type=jnp.float32, precision=_HI)
                + jnp.dot(embd[:, 0:27], w[0, _W4 + 32:_W4 + 59, 0:32],
                          preferred_element_type=jnp.float32, precision=_HI)
                + w[0, _B4:_B4 + 1, 0:32])
            c8 = jax.nn.sigmoid(
                jnp.dot(h4, w[0, _W5:_W5 + 32, 0:8],
                        preferred_element_type=jnp.float32, precision=_HI)
                + w[0, _B5:_B5 + 1, 0:8])
            out_ref[u] = c8 * ct[15:16, 0:8] + density * ct[15:16, 8:16]


@jax.jit
def kernel(x, d, layer1_w, layer1_b, layer2_w, layer2_b, layer3_w, layer3_b,
           layer4_w, layer4_b, layer5_w, layer5_b):
    B = x.shape[0]
    T_MAX = -(-(E + B // P) // U) * U

    mask = ((jnp.abs(x[:, 0]) < SCALE / 2) & (jnp.abs(x[:, 1]) < SCALE / 2)
            & (jnp.abs(x[:, 2]) < SCALE / 2))
    idx = jnp.clip((x / (SCALE / N) + N / 2).astype(jnp.int32), 0, N - 1)
    e = (idx[:, 0] * N + idx[:, 1]) * N + idx[:, 2]
    key = jnp.where(mask, e, E)
    order = jnp.argsort(key)
    i = jnp.arange(B, dtype=jnp.int32)
    inv = jnp.zeros((B,), jnp.int32).at[order].set(i)
    kc = jnp.clip(key, 0, E - 1)
    counts = jnp.bincount(key, length=E + 1)[:E]
    off = jnp.concatenate([jnp.zeros((1,), counts.dtype), jnp.cumsum(counts)])[:E]
    ntiles = (counts + P - 1) // P
    tbase = jnp.concatenate([jnp.zeros((1,), ntiles.dtype), jnp.cumsum(ntiles)])
    t_real = tbase[E]
    tbase = tbase[:E]

    r = inv - off[kc].astype(jnp.int32)
    t_of = tbase[kc].astype(jnp.int32) + r // P
    point_slot = jnp.where(mask, t_of * P + r % P, T_MAX * P)

    starts = mask & (r % P == 0)
    scat = jnp.where(starts, t_of, T_MAX)
    meta = jnp.stack([e, jnp.minimum(counts[kc].astype(jnp.int32) - r, P)], axis=1)
    tile_meta = jnp.zeros((T_MAX, 2), jnp.int32).at[scat].set(meta, mode='drop')
    tile_count = tile_meta[:, 1]
    last_e = jnp.maximum(jnp.max(jnp.where(mask, e, -1)), 0).astype(jnp.int32)
    in_real = jnp.arange(T_MAX) < t_real
    tile_expert = jnp.where(in_real, tile_meta[:, 0], last_e)
    step_count = jnp.max(tile_count.reshape(T_MAX // U, U), axis=1)

    xd = jnp.concatenate([x, d], axis=1)
    xd_tiles = jnp.zeros((T_MAX * P + 1, 6), x.dtype).at[point_slot].set(
        xd, mode='drop')[:T_MAX * P].reshape(T_MAX, P, 6)

    def pad33(a, rows):
        return jnp.pad(a.reshape(E, rows, -1),
                       ((0, 0), (0, 0), (0, 33 - a.shape[-1])))

    w_all = jnp.concatenate([
        pad33(layer1_w, 63), jnp.zeros((E, 1, 33), jnp.float32), pad33(layer1_b, 1),
        pad33(layer2_w, 32), pad33(layer2_b, 1),
        pad33(layer3_w, 32), pad33(layer3_b, 1),
        pad33(layer4_w, 59), pad33(layer4_b, 1),
        pad33(layer5_w, 32), pad33(layer5_b, 1),
    ], axis=1)

    def wmap(u):
        return lambda s, te, tc: (te[s * U + u], 0, 0)

    grid_spec = pltpu.PrefetchScalarGridSpec(
        num_scalar_prefetch=2,
        grid=(T_MAX // U,),
        in_specs=[
            pl.BlockSpec((16, 64), lambda s, te, tc: (0, 0)),
            pl.BlockSpec((U, P, 6), lambda s, te, tc: (s, 0, 0)),
        ] + [pl.BlockSpec((1, _RT, 33), wmap(u)) for u in range(U)],
        out_specs=pl.BlockSpec((U, P, 8), lambda s, te, tc: (s, 0, 0)),
    )
    out_tiles = pl.pallas_call(
        _mlp_body,
        grid_spec=grid_spec,
        out_shape=jax.ShapeDtypeStruct((T_MAX, P, 8), jnp.float32),
    )(tile_expert, step_count, jnp.asarray(_CT), xd_tiles,
      *([w_all] * U))

    res = out_tiles.reshape(T_MAX * P, 8)[point_slot]
    color = jnp.where(mask[:, None], res[:, 0:3], 0.0)
    sigma = jnp.where(mask, res[:, 3], 0.0)
    return (color, sigma)


# speedup vs baseline: 1.7205x; 1.0516x over previous
import jax
import jax.numpy as jnp
import numpy as np
from jax.experimental import pallas as pl
from jax.experimental.pallas import tpu as pltpu

N = 16
SCALE = 3.0
LP = 10
LD = 4
E = N * N * N
P = 32
U = 4

_W1, _B1 = 0, 64
_W2, _B2 = 65, 97
_W3, _B3 = 98, 130
_W4, _B4 = 131, 190
_W5, _B5 = 191, 223
_RT = 224


def _enc_consts(ncols, L):
    k = np.arange(ncols)
    g = (k - 3) // 6
    c = (k - 3) % 6
    valid = (k >= 3) & (k < 3 + 6 * L)
    scale = np.where(valid, 2.0 ** np.maximum(g, 0), 0.0)
    mid = (k < 3).astype(np.float32)
    msin = (valid & (c < 3)).astype(np.float32)
    mcos = (valid & (c >= 3)).astype(np.float32)
    sel = np.zeros((3, ncols), np.float32)
    sel[k % 3, k] = 1.0
    return sel, scale.astype(np.float32), mid, msin.astype(np.float32), mcos.astype(np.float32)


def _pack_consts():
    sx, scx, midx, msinx, mcosx = _enc_consts(64, LP)
    sd, scd, midd, msind, mcosd = _enc_consts(32, LD)
    ct = np.zeros((16, 64), np.float32)
    ct[0:3, :] = sx
    ct[3, :] = scx
    ct[4, :] = midx
    ct[5, :] = msinx
    ct[6, :] = mcosx
    ct[8:11, 0:32] = sd
    ct[11, 0:32] = scd
    ct[12, 0:32] = midd
    ct[13, 0:32] = msind
    ct[14, 0:32] = mcosd
    ct[15, 0:8] = [1, 1, 1, 0, 0, 0, 0, 0]
    ct[15, 8:16] = [0, 0, 0, 1, 0, 0, 0, 0]
    return ct


_CT = _pack_consts()
_TWO_PI = float(2.0 * np.pi)
_HI = jax.lax.Precision.HIGHEST

_SC1 = (0.9999997069576263, -0.16666577198092575, 0.008332557998438019,
        -0.0001981257223825244, 2.7040473314678126e-06, -2.05340800751852e-08)
_CC1 = (0.99999944367877, -0.49999558165608393, 0.04166103279016802,
        -0.0013862747315870928, 2.4253192495701792e-05, -2.2193949933413393e-07)


def _rr(a):
    y = a * (1.0 / _TWO_PI)
    return (y - jnp.round(y)) * _TWO_PI


def _sinp(y):
    t = y * y
    c = _SC1
    return y * (c[0] + t * (c[1] + t * (c[2] + t * (c[3] + t * (c[4] + t * c[5])))))


def _cosp(y):
    t = y * y
    c = _CC1
    return c[0] + t * (c[1] + t * (c[2] + t * (c[3] + t * (c[4] + t * c[5]))))


_LOGP = 5


def _route_body(key_ref, pslot_ref, te_ref, tcnt_ref, treal_ref,
                cnt_ref, seen_ref, tb_ref):
    nB = pslot_ref.shape[0]
    nT = te_ref.shape[0]

    def z(a, _):
        cnt_ref[a] = 0
        seen_ref[a] = 0
        return _
    jax.lax.fori_loop(0, E + 1, z, None)

    def c1(q, _):
        cnt_ref[key_ref[q]] += 1
        return _
    jax.lax.fori_loop(0, nB, c1, None)

    def c2(a, tb):
        tb_ref[a] = tb
        return tb + ((cnt_ref[a] + P - 1) >> _LOGP)
    t_real = jax.lax.fori_loop(0, E, c2, 0)
    treal_ref[0] = t_real

    def c4(q, _):
        k = key_ref[q]
        kc = jnp.minimum(k, E - 1)
        r = seen_ref[k]
        seen_ref[k] = r + 1
        t = tb_ref[kc] + (r >> _LOGP)
        valid = k < E
        pslot_ref[q] = jnp.where(valid, (t << _LOGP) + (r & (P - 1)), nT * P)

        @pl.when(valid & ((r & (P - 1)) == 0))
        def _():
            te_ref[t] = k
            tcnt_ref[t] = jnp.minimum(cnt_ref[k] - r, P)
        return _
    jax.lax.fori_loop(0, nB, c4, None)

    last = jnp.where(t_real > 0, te_ref[jnp.maximum(t_real - 1, 0)], 0)

    def c5(t, _):
        te_ref[t] = last
        tcnt_ref[t] = 0
        return _
    jax.lax.fori_loop(t_real, nT, c5, None)


def _route(key, T_MAX):
    B = key.shape[0]
    grid_spec = pltpu.PrefetchScalarGridSpec(
        num_scalar_prefetch=1,
        grid=(1,),
        in_specs=[],
        out_specs=[
            pl.BlockSpec(memory_space=pltpu.SMEM),
            pl.BlockSpec(memory_space=pltpu.SMEM),
            pl.BlockSpec(memory_space=pltpu.SMEM),
            pl.BlockSpec(memory_space=pltpu.SMEM),
        ],
        scratch_shapes=[
            pltpu.SMEM((E + 1,), jnp.int32),
            pltpu.SMEM((E + 1,), jnp.int32),
            pltpu.SMEM((E,), jnp.int32),
        ],
    )
    return pl.pallas_call(
        _route_body,
        grid_spec=grid_spec,
        out_shape=[
            jax.ShapeDtypeStruct((B,), jnp.int32),
            jax.ShapeDtypeStruct((T_MAX,), jnp.int32),
            jax.ShapeDtypeStruct((T_MAX,), jnp.int32),
            jax.ShapeDtypeStruct((1,), jnp.int32),
        ],
    )(key)


def _mlp_body(te_ref, tc_ref, ct_ref, xt_ref, *rest):
    w_refs = rest[:U]
    out_ref = rest[U]
    s = pl.program_id(0)
    ct = ct_ref[...]

    @pl.when(tc_ref[s] > 0)
    def _():
        for u in range(U):
            w = w_refs[u]
            xd = xt_ref[u]
            xb = xd[:, 0:3]
            db = xd[:, 3:6]

            X = jnp.dot(xb, ct[0:3, :], preferred_element_type=jnp.float32,
                        precision=_HI)
            A = _rr(X * ct[3:4, :])
            embx = ct[4:5, :] * X + ct[5:6, :] * _sinp(A) + ct[6:7, :] * _cosp(A)

            D = jnp.dot(db, ct[8:11, 0:32], preferred_element_type=jnp.float32,
                        precision=_HI)
            Ad = _rr(D * ct[11:12, 0:32])
            embd = (ct[12:13, 0:32] * D + ct[13:14, 0:32] * _sinp(Ad)
                    + ct[14:15, 0:32] * _cosp(Ad))

            h1 = jax.nn.relu(
                jnp.dot(embx, w[0, _W1:_W1 + 64, 0:32],
                        preferred_element_type=jnp.float32, precision=_HI)
                + w[0, _B1:_B1 + 1, 0:32])
            h2 = jax.nn.relu(
                jnp.dot(h1, w[0, _W2:_W2 + 32, 0:33],
                        preferred_element_type=jnp.float32, precision=_HI)
                + w[0, _B2:_B2 + 1, 0:33])
            density = h2[:, 32:33]
            h3 = (jnp.dot(h2[:, 0:32], w[0, _W3:_W3 + 32, 0:32],
                          preferred_element_type=jnp.float32, precision=_HI)
                  + w[0, _B3:_B3 + 1, 0:32])
            h4 = jax.nn.relu(
                jnp.dot(h3, w[0, _W4:_W4 + 32, 0:32],
                        preferred_element_type=jnp.float32, precision=_HI)
                + jnp.dot(embd[:, 0:27], w[0, _W4 + 32:_W4 + 59, 0:32],
                          preferred_element_type=jnp.float32, precision=_HI)
                + w[0, _B4:_B4 + 1, 0:32])
            c8 = jax.nn.sigmoid(
                jnp.dot(h4, w[0, _W5:_W5 + 32, 0:8],
                        preferred_element_type=jnp.float32, precision=_HI)
                + w[0, _B5:_B5 + 1, 0:8])
            out_ref[u] = c8 * ct[15:16, 0:8] + density * ct[15:16, 8:16]


@jax.jit
def kernel(x, d, layer1_w, layer1_b, layer2_w, layer2_b, layer3_w, layer3_b,
           layer4_w, layer4_b, layer5_w, layer5_b):
    B = x.shape[0]
    T_MAX = -(-(E + B // P) // U) * U

    mask = ((jnp.abs(x[:, 0]) < SCALE / 2) & (jnp.abs(x[:, 1]) < SCALE / 2)
            & (jnp.abs(x[:, 2]) < SCALE / 2))
    idx = jnp.clip((x / (SCALE / N) + N / 2).astype(jnp.int32), 0, N - 1)
    e = (idx[:, 0] * N + idx[:, 1]) * N + idx[:, 2]
    key = jnp.where(mask, e, E)
    point_slot, tile_expert, tile_count, t_real = _route(key, T_MAX)
    step_count = (jnp.arange(T_MAX // U, dtype=jnp.int32) * U
                  < t_real[0]).astype(jnp.int32)

    xd = jnp.concatenate([x, d], axis=1)
    xd_tiles = jnp.zeros((T_MAX * P + 1, 6), x.dtype).at[point_slot].set(
        xd, mode='drop')[:T_MAX * P].reshape(T_MAX, P, 6)

    def pad33(a, rows):
        return jnp.pad(a.reshape(E, rows, -1),
                       ((0, 0), (0, 0), (0, 33 - a.shape[-1])))

    w_all = jnp.concatenate([
        pad33(layer1_w, 63), jnp.zeros((E, 1, 33), jnp.float32), pad33(layer1_b, 1),
        pad33(layer2_w, 32), pad33(layer2_b, 1),
        pad33(layer3_w, 32), pad33(layer3_b, 1),
        pad33(layer4_w, 59), pad33(layer4_b, 1),
        pad33(layer5_w, 32), pad33(layer5_b, 1),
    ], axis=1)

    def wmap(u):
        return lambda s, te, tc: (te[s * U + u], 0, 0)

    grid_spec = pltpu.PrefetchScalarGridSpec(
        num_scalar_prefetch=2,
        grid=(T_MAX // U,),
        in_specs=[
            pl.BlockSpec((16, 64), lambda s, te, tc: (0, 0)),
            pl.BlockSpec((U, P, 6), lambda s, te, tc: (s, 0, 0)),
        ] + [pl.BlockSpec((1, _RT, 33), wmap(u)) for u in range(U)],
        out_specs=pl.BlockSpec((U, P, 8), lambda s, te, tc: (s, 0, 0)),
    )
    out_tiles = pl.pallas_call(
        _mlp_body,
        grid_spec=grid_spec,
        out_shape=jax.ShapeDtypeStruct((T_MAX, P, 8), jnp.float32),
    )(tile_expert, step_count, jnp.asarray(_CT), xd_tiles,
      *([w_all] * U))

    res = out_tiles.reshape(T_MAX * P, 8)[point_slot]
    color = jnp.where(mask[:, None], res[:, 0:3], 0.0)
    sigma = jnp.where(mask, res[:, 3], 0.0)
    return (color, sigma)


# speedup vs baseline: 1.7718x; 1.0298x over previous
import jax
import jax.numpy as jnp
import numpy as np
from jax.experimental import pallas as pl
from jax.experimental.pallas import tpu as pltpu

N = 16
SCALE = 3.0
LP = 10
LD = 4
E = N * N * N
P = 32
U = 8

_W1, _B1 = 0, 64
_W2, _B2 = 65, 97
_W3, _B3 = 98, 130
_W4, _B4 = 131, 190
_W5, _B5 = 191, 223
_W2D, _B2D = 224, 225
_RT = 232


def _enc_consts(ncols, L):
    k = np.arange(ncols)
    g = (k - 3) // 6
    c = (k - 3) % 6
    valid = (k >= 3) & (k < 3 + 6 * L)
    scale = np.where(valid, 2.0 ** np.maximum(g, 0), 0.0)
    mid = (k < 3).astype(np.float32)
    msin = (valid & (c < 3)).astype(np.float32)
    mcos = (valid & (c >= 3)).astype(np.float32)
    sel = np.zeros((3, ncols), np.float32)
    sel[k % 3, k] = 1.0
    return sel, scale.astype(np.float32), mid, msin.astype(np.float32), mcos.astype(np.float32)


def _pack_consts():
    sx, scx, midx, msinx, mcosx = _enc_consts(64, LP)
    sd, scd, midd, msind, mcosd = _enc_consts(32, LD)
    ct = np.zeros((16, 64), np.float32)
    ct[0:3, :] = sx
    ct[3, :] = scx
    ct[4, :] = midx
    ct[5, :] = msinx
    ct[6, :] = mcosx
    ct[8:11, 0:32] = sd
    ct[11, 0:32] = scd
    ct[12, 0:32] = midd
    ct[13, 0:32] = msind
    ct[14, 0:32] = mcosd
    ct[15, 0:8] = [1, 1, 1, 0, 0, 0, 0, 0]
    ct[15, 8:16] = [0, 0, 0, 1, 0, 0, 0, 0]
    return ct


_CT = _pack_consts()
_TWO_PI = float(2.0 * np.pi)
_HI = jax.lax.Precision.HIGHEST

_SC1 = (0.9999997069576263, -0.16666577198092575, 0.008332557998438019,
        -0.0001981257223825244, 2.7040473314678126e-06, -2.05340800751852e-08)
_CC1 = (0.99999944367877, -0.49999558165608393, 0.04166103279016802,
        -0.0013862747315870928, 2.4253192495701792e-05, -2.2193949933413393e-07)


def _rr(a):
    y = a * (1.0 / _TWO_PI)
    return (y - jnp.round(y)) * _TWO_PI


def _sinp(y):
    t = y * y
    c = _SC1
    return y * (c[0] + t * (c[1] + t * (c[2] + t * (c[3] + t * (c[4] + t * c[5])))))


def _cosp(y):
    t = y * y
    c = _CC1
    return c[0] + t * (c[1] + t * (c[2] + t * (c[3] + t * (c[4] + t * c[5]))))


_LOGP = 5


def _route_body(key_ref, pslot_ref, te_ref, tcnt_ref, treal_ref,
                cnt_ref, seen_ref, tb_ref):
    nB = pslot_ref.shape[0]
    nT = te_ref.shape[0]

    def z(a, _):
        cnt_ref[a] = 0
        seen_ref[a] = 0
        return _
    jax.lax.fori_loop(0, E + 1, z, None)

    def c1(q, _):
        cnt_ref[key_ref[q]] += 1
        return _
    jax.lax.fori_loop(0, nB, c1, None)

    def c2(a, tb):
        tb_ref[a] = tb
        return tb + ((cnt_ref[a] + P - 1) >> _LOGP)
    t_real = jax.lax.fori_loop(0, E, c2, 0)
    treal_ref[0] = t_real

    def c4(q, _):
        k = key_ref[q]
        kc = jnp.minimum(k, E - 1)
        r = seen_ref[k]
        seen_ref[k] = r + 1
        t = tb_ref[kc] + (r >> _LOGP)
        valid = k < E
        pslot_ref[q] = jnp.where(valid, (t << _LOGP) + (r & (P - 1)), nT * P)

        @pl.when(valid & ((r & (P - 1)) == 0))
        def _():
            te_ref[t] = k
            tcnt_ref[t] = jnp.minimum(cnt_ref[k] - r, P)
        return _
    jax.lax.fori_loop(0, nB, c4, None)

    last = jnp.where(t_real > 0, te_ref[jnp.maximum(t_real - 1, 0)], 0)

    def c5(t, _):
        te_ref[t] = last
        tcnt_ref[t] = 0
        return _
    jax.lax.fori_loop(t_real, nT, c5, None)


def _route(key, T_MAX):
    B = key.shape[0]
    grid_spec = pltpu.PrefetchScalarGridSpec(
        num_scalar_prefetch=1,
        grid=(1,),
        in_specs=[],
        out_specs=[
            pl.BlockSpec(memory_space=pltpu.SMEM),
            pl.BlockSpec(memory_space=pltpu.SMEM),
            pl.BlockSpec(memory_space=pltpu.SMEM),
            pl.BlockSpec(memory_space=pltpu.SMEM),
        ],
        scratch_shapes=[
            pltpu.SMEM((E + 1,), jnp.int32),
            pltpu.SMEM((E + 1,), jnp.int32),
            pltpu.SMEM((E,), jnp.int32),
        ],
    )
    return pl.pallas_call(
        _route_body,
        grid_spec=grid_spec,
        out_shape=[
            jax.ShapeDtypeStruct((B,), jnp.int32),
            jax.ShapeDtypeStruct((T_MAX,), jnp.int32),
            jax.ShapeDtypeStruct((T_MAX,), jnp.int32),
            jax.ShapeDtypeStruct((1,), jnp.int32),
        ],
    )(key)


def _mlp_body(te_ref, tc_ref, ct_ref, xt_ref, *rest):
    w_refs = rest[:U]
    out_ref = rest[U]
    s = pl.program_id(0)
    ct = ct_ref[...]

    @pl.when(tc_ref[s] > 0)
    def _():
        for u in range(U):
            w = w_refs[u]
            xd = xt_ref[u]
            xb = xd[:, 0:3]
            db = xd[:, 3:6]

            X = jnp.dot(xb, ct[0:3, :], preferred_element_type=jnp.float32,
                        precision=_HI)
            A = _rr(X * ct[3:4, :])
            embx = ct[4:5, :] * X + ct[5:6, :] * _sinp(A) + ct[6:7, :] * _cosp(A)

            D = jnp.dot(db, ct[8:11, 0:32], preferred_element_type=jnp.float32,
                        precision=_HI)
            Ad = _rr(D * ct[11:12, 0:32])
            embd = (ct[12:13, 0:32] * D + ct[13:14, 0:32] * _sinp(Ad)
                    + ct[14:15, 0:32] * _cosp(Ad))

            h1 = jax.nn.relu(
                jnp.dot(embx, w[0, _W1:_W1 + 64, 0:32],
                        preferred_element_type=jnp.float32, precision=_HI)
                + w[0, _B1:_B1 + 1, 0:32])
            h2 = jax.nn.relu(
                jnp.dot(h1, w[0, _W2:_W2 + 32, 0:32],
                        preferred_element_type=jnp.float32, precision=_HI)
                + w[0, _B2:_B2 + 1, 0:32])
            density = jax.nn.relu(
                jnp.sum(h1 * w[0, _W2D:_W2D + 1, 0:32], axis=1, keepdims=True)
                + w[0, _B2D:_B2D + 1, 0:1])
            h3 = (jnp.dot(h2, w[0, _W3:_W3 + 32, 0:32],
                          preferred_element_type=jnp.float32, precision=_HI)
                  + w[0, _B3:_B3 + 1, 0:32])
            h4 = jax.nn.relu(
                jnp.dot(h3, w[0, _W4:_W4 + 32, 0:32],
                        preferred_element_type=jnp.float32, precision=_HI)
                + jnp.dot(embd[:, 0:27], w[0, _W4 + 32:_W4 + 59, 0:32],
                          preferred_element_type=jnp.float32, precision=_HI)
                + w[0, _B4:_B4 + 1, 0:32])
            c8 = jax.nn.sigmoid(
                jnp.dot(h4, w[0, _W5:_W5 + 32, 0:8],
                        preferred_element_type=jnp.float32, precision=_HI)
                + w[0, _B5:_B5 + 1, 0:8])
            out_ref[u] = c8 * ct[15:16, 0:8] + density * ct[15:16, 8:16]


@jax.jit
def kernel(x, d, layer1_w, layer1_b, layer2_w, layer2_b, layer3_w, layer3_b,
           layer4_w, layer4_b, layer5_w, layer5_b):
    B = x.shape[0]
    T_MAX = -(-(E + B // P) // U) * U

    mask = ((jnp.abs(x[:, 0]) < SCALE / 2) & (jnp.abs(x[:, 1]) < SCALE / 2)
            & (jnp.abs(x[:, 2]) < SCALE / 2))
    idx = jnp.clip((x / (SCALE / N) + N / 2).astype(jnp.int32), 0, N - 1)
    e = (idx[:, 0] * N + idx[:, 1]) * N + idx[:, 2]
    key = jnp.where(mask, e, E)
    point_slot, tile_expert, tile_count, t_real = _route(key, T_MAX)
    step_count = (jnp.arange(T_MAX // U, dtype=jnp.int32) * U
                  < t_real[0]).astype(jnp.int32)

    xd = jnp.concatenate([x, d, jnp.zeros((B, 2), x.dtype)], axis=1)
    xd_tiles = jnp.zeros((T_MAX * P + 1, 8), x.dtype).at[point_slot].set(
        xd, mode='drop')[:T_MAX * P].reshape(T_MAX, P, 8)

    def pad32(a, rows):
        return jnp.pad(a.reshape(E, rows, -1),
                       ((0, 0), (0, 0), (0, 32 - a.shape[-1])))

    w2 = layer2_w.reshape(E, 32, 33)
    b2 = layer2_b.reshape(E, 1, 33)
    w_all = jnp.concatenate([
        pad32(layer1_w, 63), jnp.zeros((E, 1, 32), jnp.float32), pad32(layer1_b, 1),
        w2[:, :, 0:32], b2[:, :, 0:32],
        pad32(layer3_w, 32), pad32(layer3_b, 1),
        pad32(layer4_w, 59), pad32(layer4_b, 1),
        pad32(layer5_w, 32), pad32(layer5_b, 1),
        jnp.swapaxes(w2[:, :, 32:33], 1, 2),
        pad32(b2[:, :, 32:33], 1),
        jnp.zeros((E, _RT - 226, 32), jnp.float32),
    ], axis=1)

    def wmap(u):
        return lambda s, te, tc: (te[s * U + u], 0, 0)

    grid_spec = pltpu.PrefetchScalarGridSpec(
        num_scalar_prefetch=2,
        grid=(T_MAX // U,),
        in_specs=[
            pl.BlockSpec((16, 64), lambda s, te, tc: (0, 0)),
            pl.BlockSpec((U, P, 8), lambda s, te, tc: (s, 0, 0)),
        ] + [pl.BlockSpec((1, _RT, 32), wmap(u)) for u in range(U)],
        out_specs=pl.BlockSpec((U, P, 8), lambda s, te, tc: (s, 0, 0)),
    )
    out_tiles = pl.pallas_call(
        _mlp_body,
        grid_spec=grid_spec,
        out_shape=jax.ShapeDtypeStruct((T_MAX, P, 8), jnp.float32),
    )(tile_expert, step_count, jnp.asarray(_CT), xd_tiles,
      *([w_all] * U))

    res = out_tiles.reshape(T_MAX * P, 8)[point_slot]
    color = jnp.where(mask[:, None], res[:, 0:3], 0.0)
    sigma = jnp.where(mask, res[:, 3], 0.0)
    return (color, sigma)


# speedup vs baseline: 2.6931x; 1.5200x over previous
import jax
import jax.numpy as jnp
import numpy as np
from jax.experimental import pallas as pl
from jax.experimental.pallas import tpu as pltpu

N = 16
SCALE = 3.0
LP = 10
LD = 4
E = N * N * N
P = 32
U = 8

_W1, _B1 = 0, 64
_W2, _B2 = 65, 97
_W3, _B3 = 98, 130
_W4, _B4 = 131, 190
_W5, _B5 = 191, 223
_W2D, _B2D = 224, 225
_RT = 232


def _enc_consts(ncols, L):
    k = np.arange(ncols)
    g = (k - 3) // 6
    c = (k - 3) % 6
    valid = (k >= 3) & (k < 3 + 6 * L)
    scale = np.where(valid, 2.0 ** np.maximum(g, 0), 0.0)
    mid = (k < 3).astype(np.float32)
    msin = (valid & (c < 3)).astype(np.float32)
    mcos = (valid & (c >= 3)).astype(np.float32)
    sel = np.zeros((3, ncols), np.float32)
    sel[k % 3, k] = 1.0
    return sel, scale.astype(np.float32), mid, msin.astype(np.float32), mcos.astype(np.float32)


def _pack_consts():
    sx, scx, midx, msinx, mcosx = _enc_consts(64, LP)
    sd, scd, midd, msind, mcosd = _enc_consts(32, LD)
    ct = np.zeros((16, 64), np.float32)
    ct[0:3, :] = sx
    ct[3, :] = scx
    ct[4, :] = midx
    ct[5, :] = msinx
    ct[6, :] = mcosx
    ct[8:11, 0:32] = sd
    ct[11, 0:32] = scd
    ct[12, 0:32] = midd
    ct[13, 0:32] = msind
    ct[14, 0:32] = mcosd
    ct[15, 0:8] = [1, 1, 1, 0, 0, 0, 0, 0]
    ct[15, 8:16] = [0, 0, 0, 1, 0, 0, 0, 0]
    return ct


_CT = _pack_consts()
_TWO_PI = float(2.0 * np.pi)
_HI = jax.lax.Precision.HIGHEST

_SC1 = (0.9999997069576263, -0.16666577198092575, 0.008332557998438019,
        -0.0001981257223825244, 2.7040473314678126e-06, -2.05340800751852e-08)
_CC1 = (0.99999944367877, -0.49999558165608393, 0.04166103279016802,
        -0.0013862747315870928, 2.4253192495701792e-05, -2.2193949933413393e-07)


def _rr(a):
    y = a * (1.0 / _TWO_PI)
    return (y - jnp.round(y)) * _TWO_PI


def _sinp(y):
    t = y * y
    c = _SC1
    return y * (c[0] + t * (c[1] + t * (c[2] + t * (c[3] + t * (c[4] + t * c[5])))))


def _cosp(y):
    t = y * y
    c = _CC1
    return c[0] + t * (c[1] + t * (c[2] + t * (c[3] + t * (c[4] + t * c[5]))))


_LOGP = 5


def _route_body(key_ref, pslot_ref, te_ref, tcnt_ref, treal_ref,
                cnt_ref, seen_ref, tb_ref):
    nB = pslot_ref.shape[0]
    nT = te_ref.shape[0]

    def z(a, _):
        cnt_ref[a] = 0
        seen_ref[a] = 0
        return _
    jax.lax.fori_loop(0, E + 1, z, None)

    def c1(q, _):
        cnt_ref[key_ref[q]] += 1
        return _
    jax.lax.fori_loop(0, nB, c1, None)

    def c2(a, tb):
        tb_ref[a] = tb
        return tb + ((cnt_ref[a] + P - 1) >> _LOGP)
    t_real = jax.lax.fori_loop(0, E, c2, 0)
    treal_ref[0] = t_real

    def c4(q, _):
        k = key_ref[q]
        kc = jnp.minimum(k, E - 1)
        r = seen_ref[k]
        seen_ref[k] = r + 1
        t = tb_ref[kc] + (r >> _LOGP)
        valid = k < E
        pslot_ref[q] = jnp.where(valid, (t << _LOGP) + (r & (P - 1)), nT * P)

        @pl.when(valid & ((r & (P - 1)) == 0))
        def _():
            te_ref[t] = k
            tcnt_ref[t] = jnp.minimum(cnt_ref[k] - r, P)
        return _
    jax.lax.fori_loop(0, nB, c4, None)

    last = jnp.where(t_real > 0, te_ref[jnp.maximum(t_real - 1, 0)], 0)

    def c5(t, _):
        te_ref[t] = last
        tcnt_ref[t] = 0
        return _
    jax.lax.fori_loop(t_real, nT, c5, None)


def _route(key, T_MAX):
    B = key.shape[0]
    grid_spec = pltpu.PrefetchScalarGridSpec(
        num_scalar_prefetch=1,
        grid=(1,),
        in_specs=[],
        out_specs=[
            pl.BlockSpec(memory_space=pltpu.SMEM),
            pl.BlockSpec(memory_space=pltpu.SMEM),
            pl.BlockSpec(memory_space=pltpu.SMEM),
            pl.BlockSpec(memory_space=pltpu.SMEM),
        ],
        scratch_shapes=[
            pltpu.SMEM((E + 1,), jnp.int32),
            pltpu.SMEM((E + 1,), jnp.int32),
            pltpu.SMEM((E,), jnp.int32),
        ],
    )
    return pl.pallas_call(
        _route_body,
        grid_spec=grid_spec,
        out_shape=[
            jax.ShapeDtypeStruct((B,), jnp.int32),
            jax.ShapeDtypeStruct((T_MAX,), jnp.int32),
            jax.ShapeDtypeStruct((T_MAX,), jnp.int32),
            jax.ShapeDtypeStruct((1,), jnp.int32),
        ],
    )(key)


def _mlp_body(te_ref, tc_ref, ct_ref, xt_ref, *rest):
    w_refs = rest[:U]
    out_ref = rest[U]
    s = pl.program_id(0)
    ct = ct_ref[...]

    @pl.when(tc_ref[s] > 0)
    def _():
        def dot(a, b):
            return jnp.dot(a, b, preferred_element_type=jnp.float32,
                           precision=_HI)

        embx, embd = [], []
        for u in range(U):
            xd = xt_ref[u]
            X = dot(xd[:, 0:3], ct[0:3, :])
            A = _rr(X * ct[3:4, :])
            embx.append(ct[4:5, :] * X + ct[5:6, :] * _sinp(A)
                        + ct[6:7, :] * _cosp(A))
            D = dot(xd[:, 3:6], ct[8:11, 0:32])
            Ad = _rr(D * ct[11:12, 0:32])
            embd.append(ct[12:13, 0:32] * D + ct[13:14, 0:32] * _sinp(Ad)
                        + ct[14:15, 0:32] * _cosp(Ad))

        h1 = [jax.nn.relu(dot(embx[u], w_refs[u][0, _W1:_W1 + 64, 0:32])
                          + w_refs[u][0, _B1:_B1 + 1, 0:32]) for u in range(U)]
        h2 = [jax.nn.relu(dot(h1[u], w_refs[u][0, _W2:_W2 + 32, 0:32])
                          + w_refs[u][0, _B2:_B2 + 1, 0:32]) for u in range(U)]
        density = [jax.nn.relu(
            jnp.sum(h1[u] * w_refs[u][0, _W2D:_W2D + 1, 0:32], axis=1,
                    keepdims=True)
            + w_refs[u][0, _B2D:_B2D + 1, 0:1]) for u in range(U)]
        h3 = [dot(h2[u], w_refs[u][0, _W3:_W3 + 32, 0:32])
              + w_refs[u][0, _B3:_B3 + 1, 0:32] for u in range(U)]
        h4a = [dot(h3[u], w_refs[u][0, _W4:_W4 + 32, 0:32]) for u in range(U)]
        h4 = [jax.nn.relu(h4a[u]
                          + dot(embd[u][:, 0:27],
                                w_refs[u][0, _W4 + 32:_W4 + 59, 0:32])
                          + w_refs[u][0, _B4:_B4 + 1, 0:32]) for u in range(U)]
        c8 = [jax.nn.sigmoid(dot(h4[u], w_refs[u][0, _W5:_W5 + 32, 0:8])
                             + w_refs[u][0, _B5:_B5 + 1, 0:8]) for u in range(U)]
        for u in range(U):
            out_ref[u] = c8[u] * ct[15:16, 0:8] + density[u] * ct[15:16, 8:16]


@jax.jit
def kernel(x, d, layer1_w, layer1_b, layer2_w, layer2_b, layer3_w, layer3_b,
           layer4_w, layer4_b, layer5_w, layer5_b):
    B = x.shape[0]
    T_MAX = -(-(E + B // P) // U) * U

    mask = ((jnp.abs(x[:, 0]) < SCALE / 2) & (jnp.abs(x[:, 1]) < SCALE / 2)
            & (jnp.abs(x[:, 2]) < SCALE / 2))
    idx = jnp.clip((x / (SCALE / N) + N / 2).astype(jnp.int32), 0, N - 1)
    e = (idx[:, 0] * N + idx[:, 1]) * N + idx[:, 2]
    key = jnp.where(mask, e, E)
    point_slot, tile_expert, tile_count, t_real = _route(key, T_MAX)
    step_count = (jnp.arange(T_MAX // U, dtype=jnp.int32) * U
                  < t_real[0]).astype(jnp.int32)

    xd = jnp.concatenate([x, d, jnp.zeros((B, 2), x.dtype)], axis=1)
    xd_tiles = jnp.zeros((T_MAX * P + 1, 8), x.dtype).at[point_slot].set(
        xd, mode='drop')[:T_MAX * P].reshape(T_MAX, P, 8)

    def pad32(a, rows):
        return jnp.pad(a.reshape(E, rows, -1),
                       ((0, 0), (0, 0), (0, 32 - a.shape[-1])))

    w2 = layer2_w.reshape(E, 32, 33)
    b2 = layer2_b.reshape(E, 1, 33)
    w_all = jnp.concatenate([
        pad32(layer1_w, 63), jnp.zeros((E, 1, 32), jnp.float32), pad32(layer1_b, 1),
        w2[:, :, 0:32], b2[:, :, 0:32],
        pad32(layer3_w, 32), pad32(layer3_b, 1),
        pad32(layer4_w, 59), pad32(layer4_b, 1),
        pad32(layer5_w, 32), pad32(layer5_b, 1),
        jnp.swapaxes(w2[:, :, 32:33], 1, 2),
        pad32(b2[:, :, 32:33], 1),
        jnp.zeros((E, _RT - 226, 32), jnp.float32),
    ], axis=1)

    def wmap(u):
        return lambda s, te, tc: (te[s * U + u], 0, 0)

    grid_spec = pltpu.PrefetchScalarGridSpec(
        num_scalar_prefetch=2,
        grid=(T_MAX // U,),
        in_specs=[
            pl.BlockSpec((16, 64), lambda s, te, tc: (0, 0)),
            pl.BlockSpec((U, P, 8), lambda s, te, tc: (s, 0, 0)),
        ] + [pl.BlockSpec((1, _RT, 32), wmap(u)) for u in range(U)],
        out_specs=pl.BlockSpec((U, P, 8), lambda s, te, tc: (s, 0, 0)),
    )
    out_tiles = pl.pallas_call(
        _mlp_body,
        grid_spec=grid_spec,
        out_shape=jax.ShapeDtypeStruct((T_MAX, P, 8), jnp.float32),
    )(tile_expert, step_count, jnp.asarray(_CT), xd_tiles,
      *([w_all] * U))

    res = out_tiles.reshape(T_MAX * P, 8)[point_slot]
    color = jnp.where(mask[:, None], res[:, 0:3], 0.0)
    sigma = jnp.where(mask, res[:, 3], 0.0)
    return (color, sigma)


# speedup vs baseline: 4.1950x; 1.5577x over previous
import jax
import jax.numpy as jnp
import numpy as np
from jax.experimental import pallas as pl
from jax.experimental.pallas import tpu as pltpu

N = 16
SCALE = 3.0
LP = 10
LD = 4
E = N * N * N
P = 32
U = 8

_W1, _B1 = 0, 64
_W2, _B2 = 65, 97
_W3, _B3 = 98, 130
_W4, _B4 = 131, 190
_W5, _B5 = 191, 223
_W2D, _B2D = 224, 225
_RT = 232


def _enc_consts(ncols, L):
    k = np.arange(ncols)
    g = (k - 3) // 6
    c = (k - 3) % 6
    valid = (k >= 3) & (k < 3 + 6 * L)
    scale = np.where(valid, 2.0 ** np.maximum(g, 0), 0.0)
    mid = (k < 3).astype(np.float32)
    msin = (valid & (c < 3)).astype(np.float32)
    mcos = (valid & (c >= 3)).astype(np.float32)
    sel = np.zeros((3, ncols), np.float32)
    sel[k % 3, k] = 1.0
    return sel, scale.astype(np.float32), mid, msin.astype(np.float32), mcos.astype(np.float32)


def _pack_consts():
    sx, scx, midx, msinx, mcosx = _enc_consts(64, LP)
    sd, scd, midd, msind, mcosd = _enc_consts(32, LD)
    ct = np.zeros((16, 64), np.float32)
    ct[0:3, :] = sx
    ct[3, :] = scx
    ct[4, :] = midx
    ct[5, :] = msinx
    ct[6, :] = mcosx
    ct[8:11, 0:32] = sd
    ct[11, 0:32] = scd
    ct[12, 0:32] = midd
    ct[13, 0:32] = msind
    ct[14, 0:32] = mcosd
    ct[15, 0:8] = [1, 1, 1, 0, 0, 0, 0, 0]
    ct[15, 8:16] = [0, 0, 0, 1, 0, 0, 0, 0]
    return ct


_CT = _pack_consts()
_TWO_PI = float(2.0 * np.pi)
_HI = jax.lax.Precision.HIGHEST

_SC1 = (0.9999997069576263, -0.16666577198092575, 0.008332557998438019,
        -0.0001981257223825244, 2.7040473314678126e-06, -2.05340800751852e-08)
_CC1 = (0.99999944367877, -0.49999558165608393, 0.04166103279016802,
        -0.0013862747315870928, 2.4253192495701792e-05, -2.2193949933413393e-07)


def _rr(a):
    y = a * (1.0 / _TWO_PI)
    return (y - jnp.round(y)) * _TWO_PI


def _sinp(y):
    t = y * y
    c = _SC1
    return y * (c[0] + t * (c[1] + t * (c[2] + t * (c[3] + t * (c[4] + t * c[5])))))


def _cosp(y):
    t = y * y
    c = _CC1
    return c[0] + t * (c[1] + t * (c[2] + t * (c[3] + t * (c[4] + t * c[5]))))


_LOGP = 5


def _route_body(key_ref, pslot_ref, te_ref, tcnt_ref, treal_ref,
                cnt_ref, seen_ref, tb_ref):
    nB = pslot_ref.shape[0]
    nT = te_ref.shape[0]

    def z(a, _):
        cnt_ref[a] = 0
        seen_ref[a] = 0
        return _
    jax.lax.fori_loop(0, E + 1, z, None)

    def c1(q, _):
        cnt_ref[key_ref[q]] += 1
        return _
    jax.lax.fori_loop(0, nB, c1, None)

    def c2(a, tb):
        tb_ref[a] = tb
        return tb + ((cnt_ref[a] + P - 1) >> _LOGP)
    t_real = jax.lax.fori_loop(0, E, c2, 0)
    treal_ref[0] = t_real

    def c4(q, _):
        k = key_ref[q]
        kc = jnp.minimum(k, E - 1)
        r = seen_ref[k]
        seen_ref[k] = r + 1
        t = tb_ref[kc] + (r >> _LOGP)
        valid = k < E
        pslot_ref[q] = jnp.where(valid, (t << _LOGP) + (r & (P - 1)), nT * P)

        @pl.when(valid & ((r & (P - 1)) == 0))
        def _():
            te_ref[t] = k
            tcnt_ref[t] = jnp.minimum(cnt_ref[k] - r, P)
        return _
    jax.lax.fori_loop(0, nB, c4, None)

    last = jnp.where(t_real > 0, te_ref[jnp.maximum(t_real - 1, 0)], 0)

    def c5(t, _):
        te_ref[t] = last
        tcnt_ref[t] = 0
        return _
    jax.lax.fori_loop(t_real, nT, c5, None)


def _route(key, T_MAX):
    B = key.shape[0]
    grid_spec = pltpu.PrefetchScalarGridSpec(
        num_scalar_prefetch=1,
        grid=(1,),
        in_specs=[],
        out_specs=[
            pl.BlockSpec(memory_space=pltpu.SMEM),
            pl.BlockSpec(memory_space=pltpu.SMEM),
            pl.BlockSpec(memory_space=pltpu.SMEM),
            pl.BlockSpec(memory_space=pltpu.SMEM),
        ],
        scratch_shapes=[
            pltpu.SMEM((E + 1,), jnp.int32),
            pltpu.SMEM((E + 1,), jnp.int32),
            pltpu.SMEM((E,), jnp.int32),
        ],
    )
    return pl.pallas_call(
        _route_body,
        grid_spec=grid_spec,
        out_shape=[
            jax.ShapeDtypeStruct((B,), jnp.int32),
            jax.ShapeDtypeStruct((T_MAX,), jnp.int32),
            jax.ShapeDtypeStruct((T_MAX,), jnp.int32),
            jax.ShapeDtypeStruct((1,), jnp.int32),
        ],
    )(key)


_G = 32


def _repack_body(w1r, b1r, w2r, b2r, w3r, b3r, w4r, b4r, w5r, b5r, w2dr, out):
    out[:, _W1:_W1 + 63, :] = w1r[...]
    out[:, 63:64, :] = jnp.zeros((_G, 1, 32), jnp.float32)
    out[:, _B1:_B1 + 1, :] = b1r[...]
    out[:, _W2:_W2 + 32, :] = w2r[:, :, 0:32]
    out[:, _B2:_B2 + 1, :] = b2r[:, :, 0:32]
    out[:, _W3:_W3 + 32, :] = w3r[...]
    out[:, _B3:_B3 + 1, :] = b3r[...]
    out[:, _W4:_W4 + 59, :] = w4r[...]
    out[:, _B4:_B4 + 1, :] = b4r[...]
    out[:, _W5:_W5 + 32, :] = w5r[...]
    out[:, _B5:_B5 + 1, :] = b5r[...]
    out[:, _W2D:_W2D + 1, :] = w2dr[...]
    b2d = jnp.zeros((_G, 1, 32), jnp.float32) + b2r[:, :, 32:33]
    out[:, _B2D:_B2D + 1, :] = b2d * (jax.lax.broadcasted_iota(
        jnp.int32, (_G, 1, 32), 2) == 0).astype(jnp.float32)
    out[:, 226:232, :] = jnp.zeros((_G, 6, 32), jnp.float32)


def _repack(w1, b1, w2, b2, w3, b3, w4, b4, w5p, b5p, w2d):
    m = lambda g: (g, 0, 0)
    return pl.pallas_call(
        _repack_body,
        grid=(E // _G,),
        in_specs=[
            pl.BlockSpec((_G, 63, 32), m), pl.BlockSpec((_G, 1, 32), m),
            pl.BlockSpec((_G, 32, 33), m), pl.BlockSpec((_G, 1, 33), m),
            pl.BlockSpec((_G, 32, 32), m), pl.BlockSpec((_G, 1, 32), m),
            pl.BlockSpec((_G, 59, 32), m), pl.BlockSpec((_G, 1, 32), m),
            pl.BlockSpec((_G, 32, 32), m), pl.BlockSpec((_G, 1, 32), m),
            pl.BlockSpec((_G, 1, 32), m),
        ],
        out_specs=pl.BlockSpec((_G, _RT, 32), m),
        out_shape=jax.ShapeDtypeStruct((E, _RT, 32), jnp.float32),
    )(w1, b1, w2, b2, w3, b3, w4, b4, w5p, b5p, w2d)


def _mlp_body(te_ref, tc_ref, ct_ref, xt_ref, *rest):
    w_refs = rest[:U]
    out_ref = rest[U]
    s = pl.program_id(0)
    ct = ct_ref[...]

    @pl.when(tc_ref[s] > 0)
    def _():
        def dot(a, b):
            return jnp.dot(a, b, preferred_element_type=jnp.float32,
                           precision=_HI)

        embx, embd = [], []
        for u in range(U):
            xd = xt_ref[u]
            X = dot(xd[:, 0:3], ct[0:3, :])
            A = _rr(X * ct[3:4, :])
            embx.append(ct[4:5, :] * X + ct[5:6, :] * _sinp(A)
                        + ct[6:7, :] * _cosp(A))
            D = dot(xd[:, 3:6], ct[8:11, 0:32])
            Ad = _rr(D * ct[11:12, 0:32])
            embd.append(ct[12:13, 0:32] * D + ct[13:14, 0:32] * _sinp(Ad)
                        + ct[14:15, 0:32] * _cosp(Ad))

        h1 = [jax.nn.relu(dot(embx[u], w_refs[u][0, _W1:_W1 + 64, 0:32])
                          + w_refs[u][0, _B1:_B1 + 1, 0:32]) for u in range(U)]
        h2 = [jax.nn.relu(dot(h1[u], w_refs[u][0, _W2:_W2 + 32, 0:32])
                          + w_refs[u][0, _B2:_B2 + 1, 0:32]) for u in range(U)]
        density = [jax.nn.relu(
            jnp.sum(h1[u] * w_refs[u][0, _W2D:_W2D + 1, 0:32], axis=1,
                    keepdims=True)
            + w_refs[u][0, _B2D:_B2D + 1, 0:1]) for u in range(U)]
        h3 = [dot(h2[u], w_refs[u][0, _W3:_W3 + 32, 0:32])
              + w_refs[u][0, _B3:_B3 + 1, 0:32] for u in range(U)]
        h4a = [dot(h3[u], w_refs[u][0, _W4:_W4 + 32, 0:32]) for u in range(U)]
        h4 = [jax.nn.relu(h4a[u]
                          + dot(embd[u][:, 0:27],
                                w_refs[u][0, _W4 + 32:_W4 + 59, 0:32])
                          + w_refs[u][0, _B4:_B4 + 1, 0:32]) for u in range(U)]
        c8 = [jax.nn.sigmoid(dot(h4[u], w_refs[u][0, _W5:_W5 + 32, 0:8])
                             + w_refs[u][0, _B5:_B5 + 1, 0:8]) for u in range(U)]
        for u in range(U):
            out_ref[u] = c8[u] * ct[15:16, 0:8] + density[u] * ct[15:16, 8:16]


@jax.jit
def kernel(x, d, layer1_w, layer1_b, layer2_w, layer2_b, layer3_w, layer3_b,
           layer4_w, layer4_b, layer5_w, layer5_b):
    B = x.shape[0]
    T_MAX = -(-(E + B // P) // U) * U

    mask = ((jnp.abs(x[:, 0]) < SCALE / 2) & (jnp.abs(x[:, 1]) < SCALE / 2)
            & (jnp.abs(x[:, 2]) < SCALE / 2))
    idx = jnp.clip((x / (SCALE / N) + N / 2).astype(jnp.int32), 0, N - 1)
    e = (idx[:, 0] * N + idx[:, 1]) * N + idx[:, 2]
    key = jnp.where(mask, e, E)
    point_slot, tile_expert, tile_count, t_real = _route(key, T_MAX)
    step_count = (jnp.arange(T_MAX // U, dtype=jnp.int32) * U
                  < t_real[0]).astype(jnp.int32)

    xd = jnp.concatenate([x, d, jnp.zeros((B, 2), x.dtype)], axis=1)
    xd_tiles = jnp.zeros((T_MAX * P + 1, 8), x.dtype).at[point_slot].set(
        xd, mode='drop')[:T_MAX * P].reshape(T_MAX, P, 8)

    w2 = layer2_w.reshape(E, 32, 33)
    b2 = layer2_b.reshape(E, 1, 33)
    w2d = jnp.swapaxes(w2[:, :, 32:33], 1, 2)
    w5p = jnp.pad(layer5_w.reshape(E, 32, 3), ((0, 0), (0, 0), (0, 29)))
    b5p = jnp.pad(layer5_b.reshape(E, 1, 3), ((0, 0), (0, 0), (0, 29)))
    w_all = _repack(layer1_w.reshape(E, 63, 32), layer1_b.reshape(E, 1, 32),
                    w2, b2, layer3_w.reshape(E, 32, 32),
                    layer3_b.reshape(E, 1, 32), layer4_w.reshape(E, 59, 32),
                    layer4_b.reshape(E, 1, 32), w5p, b5p, w2d)

    def wmap(u):
        return lambda s, te, tc: (te[s * U + u], 0, 0)

    grid_spec = pltpu.PrefetchScalarGridSpec(
        num_scalar_prefetch=2,
        grid=(T_MAX // U,),
        in_specs=[
            pl.BlockSpec((16, 64), lambda s, te, tc: (0, 0)),
            pl.BlockSpec((U, P, 8), lambda s, te, tc: (s, 0, 0)),
        ] + [pl.BlockSpec((1, _RT, 32), wmap(u)) for u in range(U)],
        out_specs=pl.BlockSpec((U, P, 8), lambda s, te, tc: (s, 0, 0)),
    )
    out_tiles = pl.pallas_call(
        _mlp_body,
        grid_spec=grid_spec,
        out_shape=jax.ShapeDtypeStruct((T_MAX, P, 8), jnp.float32),
    )(tile_expert, step_count, jnp.asarray(_CT), xd_tiles,
      *([w_all] * U))

    res = out_tiles.reshape(T_MAX * P, 8)[point_slot]
    color = jnp.where(mask[:, None], res[:, 0:3], 0.0)
    sigma = jnp.where(mask, res[:, 3], 0.0)
    return (color, sigma)


# speedup vs baseline: 4.3826x; 1.0447x over previous
import jax
import jax.numpy as jnp
import numpy as np
from jax.experimental import pallas as pl
from jax.experimental.pallas import tpu as pltpu

N = 16
SCALE = 3.0
LP = 10
LD = 4
E = N * N * N
P = 32
U = 16

_W1, _B1 = 0, 64
_W2, _B2 = 65, 97
_W3, _B3 = 98, 130
_W4, _B4 = 131, 190
_W5, _B5 = 191, 223
_W2D, _B2D = 224, 225
_RT = 232


def _enc_consts(ncols, L):
    k = np.arange(ncols)
    g = (k - 3) // 6
    c = (k - 3) % 6
    valid = (k >= 3) & (k < 3 + 6 * L)
    scale = np.where(valid, 2.0 ** np.maximum(g, 0), 0.0)
    mid = (k < 3).astype(np.float32)
    msin = (valid & (c < 3)).astype(np.float32)
    mcos = (valid & (c >= 3)).astype(np.float32)
    sel = np.zeros((3, ncols), np.float32)
    sel[k % 3, k] = 1.0
    return sel, scale.astype(np.float32), mid, msin.astype(np.float32), mcos.astype(np.float32)


def _pack_consts():
    sx, scx, midx, msinx, mcosx = _enc_consts(64, LP)
    sd, scd, midd, msind, mcosd = _enc_consts(32, LD)
    ct = np.zeros((16, 64), np.float32)
    ct[0:3, :] = sx
    ct[3, :] = scx
    ct[4, :] = midx
    ct[5, :] = msinx
    ct[6, :] = mcosx
    ct[8:11, 0:32] = sd
    ct[11, 0:32] = scd
    ct[12, 0:32] = midd
    ct[13, 0:32] = msind
    ct[14, 0:32] = mcosd
    ct[15, 0:8] = [1, 1, 1, 0, 0, 0, 0, 0]
    ct[15, 8:16] = [0, 0, 0, 1, 0, 0, 0, 0]
    return ct


_CT = _pack_consts()
_TWO_PI = float(2.0 * np.pi)
_HI = jax.lax.Precision.HIGHEST

_SC1 = (0.9999997069576263, -0.16666577198092575, 0.008332557998438019,
        -0.0001981257223825244, 2.7040473314678126e-06, -2.05340800751852e-08)
_CC1 = (0.99999944367877, -0.49999558165608393, 0.04166103279016802,
        -0.0013862747315870928, 2.4253192495701792e-05, -2.2193949933413393e-07)


def _rr(a):
    y = a * (1.0 / _TWO_PI)
    return (y - jnp.round(y)) * _TWO_PI


def _sinp(y):
    t = y * y
    c = _SC1
    return y * (c[0] + t * (c[1] + t * (c[2] + t * (c[3] + t * (c[4] + t * c[5])))))


def _cosp(y):
    t = y * y
    c = _CC1
    return c[0] + t * (c[1] + t * (c[2] + t * (c[3] + t * (c[4] + t * c[5]))))


_LOGP = 5


def _route_body(key_ref, pslot_ref, te_ref, tcnt_ref, treal_ref,
                cnt_ref, seen_ref, tb_ref):
    nB = pslot_ref.shape[0]
    nT = te_ref.shape[0]

    def z(a, _):
        cnt_ref[a] = 0
        seen_ref[a] = 0
        return _
    jax.lax.fori_loop(0, E + 1, z, None)

    def c1(q, _):
        cnt_ref[key_ref[q]] += 1
        return _
    jax.lax.fori_loop(0, nB, c1, None)

    def c2(a, tb):
        tb_ref[a] = tb
        return tb + ((cnt_ref[a] + P - 1) >> _LOGP)
    t_real = jax.lax.fori_loop(0, E, c2, 0)
    treal_ref[0] = t_real

    def c4(q, _):
        k = key_ref[q]
        kc = jnp.minimum(k, E - 1)
        r = seen_ref[k]
        seen_ref[k] = r + 1
        t = tb_ref[kc] + (r >> _LOGP)
        valid = k < E
        pslot_ref[q] = jnp.where(valid, (t << _LOGP) + (r & (P - 1)), nT * P)

        @pl.when(valid & ((r & (P - 1)) == 0))
        def _():
            te_ref[t] = k
            tcnt_ref[t] = jnp.minimum(cnt_ref[k] - r, P)
        return _
    jax.lax.fori_loop(0, nB, c4, None)

    last = jnp.where(t_real > 0, te_ref[jnp.maximum(t_real - 1, 0)], 0)

    def c5(t, _):
        te_ref[t] = last
        tcnt_ref[t] = 0
        return _
    jax.lax.fori_loop(t_real, nT, c5, None)


def _route(key, T_MAX):
    B = key.shape[0]
    grid_spec = pltpu.PrefetchScalarGridSpec(
        num_scalar_prefetch=1,
        grid=(1,),
        in_specs=[],
        out_specs=[
            pl.BlockSpec(memory_space=pltpu.SMEM),
            pl.BlockSpec(memory_space=pltpu.SMEM),
            pl.BlockSpec(memory_space=pltpu.SMEM),
            pl.BlockSpec(memory_space=pltpu.SMEM),
        ],
        scratch_shapes=[
            pltpu.SMEM((E + 1,), jnp.int32),
            pltpu.SMEM((E + 1,), jnp.int32),
            pltpu.SMEM((E,), jnp.int32),
        ],
    )
    return pl.pallas_call(
        _route_body,
        grid_spec=grid_spec,
        out_shape=[
            jax.ShapeDtypeStruct((B,), jnp.int32),
            jax.ShapeDtypeStruct((T_MAX,), jnp.int32),
            jax.ShapeDtypeStruct((T_MAX,), jnp.int32),
            jax.ShapeDtypeStruct((1,), jnp.int32),
        ],
    )(key)


_G = 32


def _repack_body(w1r, b1r, w2r, b2r, w3r, b3r, w4r, b4r, w5r, b5r, w2dr, out):
    out[:, _W1:_W1 + 63, :] = w1r[...]
    out[:, 63:64, :] = jnp.zeros((_G, 1, 32), jnp.float32)
    out[:, _B1:_B1 + 1, :] = b1r[...]
    out[:, _W2:_W2 + 32, :] = w2r[:, :, 0:32]
    out[:, _B2:_B2 + 1, :] = b2r[:, :, 0:32]
    out[:, _W3:_W3 + 32, :] = w3r[...]
    out[:, _B3:_B3 + 1, :] = b3r[...]
    out[:, _W4:_W4 + 59, :] = w4r[...]
    out[:, _B4:_B4 + 1, :] = b4r[...]
    out[:, _W5:_W5 + 32, :] = w5r[...]
    out[:, _B5:_B5 + 1, :] = b5r[...]
    out[:, _W2D:_W2D + 1, :] = w2dr[...]
    b2d = jnp.zeros((_G, 1, 32), jnp.float32) + b2r[:, :, 32:33]
    out[:, _B2D:_B2D + 1, :] = b2d * (jax.lax.broadcasted_iota(
        jnp.int32, (_G, 1, 32), 2) == 0).astype(jnp.float32)
    out[:, 226:232, :] = jnp.zeros((_G, 6, 32), jnp.float32)


def _repack(w1, b1, w2, b2, w3, b3, w4, b4, w5p, b5p, w2d):
    m = lambda g: (g, 0, 0)
    return pl.pallas_call(
        _repack_body,
        grid=(E // _G,),
        in_specs=[
            pl.BlockSpec((_G, 63, 32), m), pl.BlockSpec((_G, 1, 32), m),
            pl.BlockSpec((_G, 32, 33), m), pl.BlockSpec((_G, 1, 33), m),
            pl.BlockSpec((_G, 32, 32), m), pl.BlockSpec((_G, 1, 32), m),
            pl.BlockSpec((_G, 59, 32), m), pl.BlockSpec((_G, 1, 32), m),
            pl.BlockSpec((_G, 32, 32), m), pl.BlockSpec((_G, 1, 32), m),
            pl.BlockSpec((_G, 1, 32), m),
        ],
        out_specs=pl.BlockSpec((_G, _RT, 32), m),
        out_shape=jax.ShapeDtypeStruct((E, _RT, 32), jnp.float32),
    )(w1, b1, w2, b2, w3, b3, w4, b4, w5p, b5p, w2d)


def _mlp_body(te_ref, tc_ref, ct_ref, xt_ref, *rest):
    w_refs = rest[:U]
    out_ref = rest[U]
    s = pl.program_id(0)
    ct = ct_ref[...]

    @pl.when(tc_ref[s] > 0)
    def _():
        def dot(a, b):
            return jnp.dot(a, b, preferred_element_type=jnp.float32,
                           precision=_HI)

        embx, embd = [], []
        for u in range(U):
            xd = xt_ref[u]
            X = dot(xd[:, 0:3], ct[0:3, :])
            A = _rr(X * ct[3:4, :])
            embx.append(ct[4:5, :] * X + ct[5:6, :] * _sinp(A)
                        + ct[6:7, :] * _cosp(A))
            D = dot(xd[:, 3:6], ct[8:11, 0:32])
            Ad = _rr(D * ct[11:12, 0:32])
            embd.append(ct[12:13, 0:32] * D + ct[13:14, 0:32] * _sinp(Ad)
                        + ct[14:15, 0:32] * _cosp(Ad))

        h1 = [jax.nn.relu(dot(embx[u], w_refs[u][0, _W1:_W1 + 64, 0:32])
                          + w_refs[u][0, _B1:_B1 + 1, 0:32]) for u in range(U)]
        h2 = [jax.nn.relu(dot(h1[u], w_refs[u][0, _W2:_W2 + 32, 0:32])
                          + w_refs[u][0, _B2:_B2 + 1, 0:32]) for u in range(U)]
        density = [jax.nn.relu(
            jnp.sum(h1[u] * w_refs[u][0, _W2D:_W2D + 1, 0:32], axis=1,
                    keepdims=True)
            + w_refs[u][0, _B2D:_B2D + 1, 0:1]) for u in range(U)]
        h3 = [dot(h2[u], w_refs[u][0, _W3:_W3 + 32, 0:32])
              + w_refs[u][0, _B3:_B3 + 1, 0:32] for u in range(U)]
        h4a = [dot(h3[u], w_refs[u][0, _W4:_W4 + 32, 0:32]) for u in range(U)]
        h4 = [jax.nn.relu(h4a[u]
                          + dot(embd[u][:, 0:27],
                                w_refs[u][0, _W4 + 32:_W4 + 59, 0:32])
                          + w_refs[u][0, _B4:_B4 + 1, 0:32]) for u in range(U)]
        c8 = [jax.nn.sigmoid(dot(h4[u], w_refs[u][0, _W5:_W5 + 32, 0:8])
                             + w_refs[u][0, _B5:_B5 + 1, 0:8]) for u in range(U)]
        for u in range(U):
            out_ref[u] = c8[u] * ct[15:16, 0:8] + density[u] * ct[15:16, 8:16]


@jax.jit
def kernel(x, d, layer1_w, layer1_b, layer2_w, layer2_b, layer3_w, layer3_b,
           layer4_w, layer4_b, layer5_w, layer5_b):
    B = x.shape[0]
    T_MAX = -(-(E + B // P) // U) * U

    mask = ((jnp.abs(x[:, 0]) < SCALE / 2) & (jnp.abs(x[:, 1]) < SCALE / 2)
            & (jnp.abs(x[:, 2]) < SCALE / 2))
    idx = jnp.clip((x / (SCALE / N) + N / 2).astype(jnp.int32), 0, N - 1)
    e = (idx[:, 0] * N + idx[:, 1]) * N + idx[:, 2]
    key = jnp.where(mask, e, E)
    point_slot, tile_expert, tile_count, t_real = _route(key, T_MAX)
    step_count = (jnp.arange(T_MAX // U, dtype=jnp.int32) * U
                  < t_real[0]).astype(jnp.int32)

    xd = jnp.concatenate([x, d, jnp.zeros((B, 2), x.dtype)], axis=1)
    xd_tiles = jnp.zeros((T_MAX * P + 1, 8), x.dtype).at[point_slot].set(
        xd, mode='drop')[:T_MAX * P].reshape(T_MAX, P, 8)

    w2 = layer2_w.reshape(E, 32, 33)
    b2 = layer2_b.reshape(E, 1, 33)
    w2d = jnp.swapaxes(w2[:, :, 32:33], 1, 2)
    w5p = jnp.pad(layer5_w.reshape(E, 32, 3), ((0, 0), (0, 0), (0, 29)))
    b5p = jnp.pad(layer5_b.reshape(E, 1, 3), ((0, 0), (0, 0), (0, 29)))
    w_all = _repack(layer1_w.reshape(E, 63, 32), layer1_b.reshape(E, 1, 32),
                    w2, b2, layer3_w.reshape(E, 32, 32),
                    layer3_b.reshape(E, 1, 32), layer4_w.reshape(E, 59, 32),
                    layer4_b.reshape(E, 1, 32), w5p, b5p, w2d)

    def wmap(u):
        return lambda s, te, tc: (te[s * U + u], 0, 0)

    grid_spec = pltpu.PrefetchScalarGridSpec(
        num_scalar_prefetch=2,
        grid=(T_MAX // U,),
        in_specs=[
            pl.BlockSpec((16, 64), lambda s, te, tc: (0, 0)),
            pl.BlockSpec((U, P, 8), lambda s, te, tc: (s, 0, 0)),
        ] + [pl.BlockSpec((1, _RT, 32), wmap(u)) for u in range(U)],
        out_specs=pl.BlockSpec((U, P, 8), lambda s, te, tc: (s, 0, 0)),
    )
    out_tiles = pl.pallas_call(
        _mlp_body,
        grid_spec=grid_spec,
        out_shape=jax.ShapeDtypeStruct((T_MAX, P, 8), jnp.float32),
    )(tile_expert, step_count, jnp.asarray(_CT), xd_tiles,
      *([w_all] * U))

    res = out_tiles.reshape(T_MAX * P, 8)[point_slot]
    color = jnp.where(mask[:, None], res[:, 0:3], 0.0)
    sigma = jnp.where(mask, res[:, 3], 0.0)
    return (color, sigma)
